# baseline (speedup 1.0000x reference)
import functools

import jax
import jax.numpy as jnp
from jax.experimental import pallas as pl
from jax.experimental.pallas import tpu as pltpu

B, K, V = 32, 8, 100000
PAD_TOKEN_ID = 0
ROWS = B * (K + 1)
VB = 2048
NV = -(-V // VB)
NEG_INF = float("-inf")
IBIG = 2**31 - 1


def _scan_body(v_idx, tp_ref, ti_ref, did_ref, dp_ref, tid_out, tps_out,
               s_max, s_idx, s_ti, s_sum, s_tps):
    v = v_idx

    @pl.when(v == 0)
    def _init():
        s_max[...] = jnp.full((8, 128), NEG_INF, jnp.float32)
        s_idx[...] = jnp.zeros((8, 128), jnp.int32)
        s_ti[...] = jnp.zeros((8, 128), jnp.int32)
        s_sum[...] = jnp.zeros((8, 128), jnp.float32)
        s_tps[...] = jnp.zeros((8, 128), jnp.float32)

    tp = tp_ref[...]
    ti = ti_ref[...]
    did = did_ref[:, :1]
    dp = dp_ref[:, :1]

    gcol = jax.lax.broadcasted_iota(jnp.int32, (8, VB), 1) + v * VB
    valid = gcol < V
    eq = (ti == did) & valid
    tpv = jnp.where(valid, tp, 0.0)
    c = jnp.maximum(tpv - jnp.where(eq, dp, 0.0), 0.0)

    s_sum[...] = s_sum[...] + jnp.sum(c, axis=1, keepdims=True)
    s_tps[...] = s_tps[...] + jnp.sum(jnp.where(eq, tpv, 0.0), axis=1,
                                      keepdims=True)

    m = jnp.max(c, axis=1, keepdims=True)
    idx = jnp.min(jnp.where(c == m, gcol, IBIG), axis=1, keepdims=True)
    tiv = jnp.sum(jnp.where(gcol == idx, ti, 0), axis=1, keepdims=True)

    old = s_max[...]
    upd = m > old
    s_max[...] = jnp.where(upd, m, old)
    s_idx[...] = jnp.where(upd, idx, s_idx[...])
    s_ti[...] = jnp.where(upd, tiv, s_ti[...])

    @pl.when(v == NV - 1)
    def _emit():
        tid_out[...] = s_ti[...]
        tps_out[...] = s_tps[...]


def _scan_kernel(tp_ref, ti_ref, did_ref, dp_ref, tid_out, tps_out,
                 s_max, s_idx, s_ti, s_sum, s_tps):
    v = pl.program_id(1)
    _scan_body(v, tp_ref, ti_ref, did_ref, dp_ref, tid_out, tps_out,
               s_max, s_idx, s_ti, s_sum, s_tps)


def _tail_kernel(tid_ref, tps_ref, did_ref, dp_ref, rnd_ref,
                 tok_out, idx_out):
    tid = tid_ref[...]
    tps = tps_ref[...]
    did = did_ref[...]
    dp = dp_ref[...]
    rnd = rnd_ref[...]

    ratio = jnp.minimum(tps / dp, 1.0)
    acc = (rnd < ratio).astype(jnp.float32)

    r_i = jax.lax.broadcasted_iota(jnp.int32, (16, 16), 0)
    c_i = jax.lax.broadcasted_iota(jnp.int32, (16, 16), 1)
    lt = (r_i <= c_i).astype(jnp.float32)
    cs = jnp.dot(acc, lt, preferred_element_type=jnp.float32)

    lane = jax.lax.broadcasted_iota(jnp.int32, (32, 16), 1)
    positions = (lane + 1).astype(jnp.float32)
    mask = cs == positions
    index = jnp.sum(mask.astype(jnp.int32), axis=1, keepdims=True)

    tokens = jnp.where(mask, did, tid)
    keep = index >= lane
    tokens = jnp.where(keep, tokens, PAD_TOKEN_ID)

    tok_out[...] = tokens
    idx_out[...] = jnp.broadcast_to(index, (32, 16))


@jax.jit
def kernel(draft_ids, draft_probs, target_probs, target_indices):
    tp2d = target_probs.reshape(ROWS, V)
    ti2d = target_indices.reshape(ROWS, V)

    did_rows = jnp.concatenate(
        [draft_ids, jnp.full((B, 1), -1, jnp.int32)], axis=1).reshape(ROWS)
    dp_rows = jnp.concatenate(
        [draft_probs, jnp.zeros((B, 1), jnp.float32)], axis=1).reshape(ROWS)
    did_b = jnp.broadcast_to(did_rows[:, None], (ROWS, 128))
    dp_b = jnp.broadcast_to(dp_rows[:, None], (ROWS, 128))

    grid = (ROWS // 8, NV)
    tid_full, tps_full = pl.pallas_call(
        _scan_kernel,
        grid=grid,
        in_specs=[
            pl.BlockSpec((8, VB), lambda r, v: (r, v)),
            pl.BlockSpec((8, VB), lambda r, v: (r, v)),
            pl.BlockSpec((8, 128), lambda r, v: (r, 0)),
            pl.BlockSpec((8, 128), lambda r, v: (r, 0)),
        ],
        out_specs=[
            pl.BlockSpec((8, 128), lambda r, v: (r, 0)),
            pl.BlockSpec((8, 128), lambda r, v: (r, 0)),
        ],
        out_shape=[
            jax.ShapeDtypeStruct((ROWS, 128), jnp.int32),
            jax.ShapeDtypeStruct((ROWS, 128), jnp.float32),
        ],
        scratch_shapes=[
            pltpu.VMEM((8, 128), jnp.float32),
            pltpu.VMEM((8, 128), jnp.int32),
            pltpu.VMEM((8, 128), jnp.int32),
            pltpu.VMEM((8, 128), jnp.float32),
            pltpu.VMEM((8, 128), jnp.float32),
        ],
        compiler_params=pltpu.CompilerParams(
            dimension_semantics=("arbitrary", "arbitrary")),
    )(tp2d, ti2d, did_b, dp_b)

    target_ids = tid_full[:, 0].reshape(B, K + 1)
    tp_sel = tps_full[:, 0].reshape(B, K + 1)[:, :K]

    tid_p = jnp.pad(target_ids, ((0, 0), (0, 16 - (K + 1))))
    tps_p = jnp.pad(tp_sel, ((0, 0), (0, 16 - K)))
    did_p = jnp.pad(draft_ids, ((0, 0), (0, 16 - K)))
    dp_p = jnp.pad(draft_probs, ((0, 0), (0, 16 - K)), constant_values=1.0)
    rnd = jax.random.uniform(jax.random.key(42), (B, K), dtype=jnp.float32)
    rnd_p = jnp.pad(rnd, ((0, 0), (0, 16 - K)), constant_values=2.0)

    tokens16, idx16 = pl.pallas_call(
        _tail_kernel,
        out_shape=[
            jax.ShapeDtypeStruct((B, 16), jnp.int32),
            jax.ShapeDtypeStruct((B, 16), jnp.int32),
        ],
    )(tid_p, tps_p, did_p, dp_p, rnd_p)

    tokens = tokens16[:, :K + 1]
    index = idx16[:, :1]
    return (tokens, index)


# speedup vs baseline: 1.1479x; 1.1479x over previous
import jax
import jax.numpy as jnp
from jax.experimental import pallas as pl
from jax.experimental.pallas import tpu as pltpu

B, K, V = 32, 8, 100000
PAD_TOKEN_ID = 0
ROWS = B * (K + 1)
RB = 8
VB = 2048
NV = -(-V // VB)
IBIG = 2**31 - 1


def _scan_kernel(tp_ref, ti_ref, did_ref, dp_ref, tid_out, tps_out,
                 s_max, s_v, s_ti, s_sum, s_tps, s_ti0):
    v = pl.program_id(1)

    tp = tp_ref[...]
    ti = ti_ref[...]
    did = did_ref[:, :1]
    dp = dp_ref[:, :1]

    def pieces(tp_, eq_):
        c = jnp.maximum(tp_ - jnp.where(eq_, dp, 0.0), 0.0)
        msel = jnp.where(eq_, tp_, 0.0)
        return c, msel

    @pl.when(v == 0)
    def _init():
        eq = ti == did
        c, msel = pieces(tp, eq)
        s_max[...] = c
        s_v[...] = jnp.zeros((RB, VB), jnp.int32)
        s_ti[...] = ti
        s_sum[...] = c
        s_tps[...] = msel
        s_ti0[...] = ti[:, :128]

    @pl.when((v > 0) & (v < NV - 1))
    def _mid():
        eq = ti == did
        c, msel = pieces(tp, eq)
        s_sum[...] = s_sum[...] + c
        s_tps[...] = s_tps[...] + msel
        upd = c > s_max[...]
        s_max[...] = jnp.where(upd, c, s_max[...])
        s_v[...] = jnp.where(upd, v, s_v[...])
        s_ti[...] = jnp.where(upd, ti, s_ti[...])

    @pl.when(v == NV - 1)
    def _last():
        lane = jax.lax.broadcasted_iota(jnp.int32, (RB, VB), 1)
        valid = (lane + v * VB) < V
        eq = (ti == did) & valid
        c, msel = pieces(jnp.where(valid, tp, 0.0), eq)
        s_sum[...] = s_sum[...] + c
        s_tps[...] = s_tps[...] + msel
        upd = c > s_max[...]
        smax = jnp.where(upd, c, s_max[...])
        sv = jnp.where(upd, v, s_v[...])
        sti = jnp.where(upd, ti, s_ti[...])

        m = jnp.max(smax, axis=1, keepdims=True)
        colg = sv * VB + lane
        idx = jnp.min(jnp.where(smax == m, colg, IBIG), axis=1,
                      keepdims=True)
        tiv = jnp.sum(jnp.where(colg == idx, sti, 0), axis=1,
                      keepdims=True)
        ssum = jnp.sum(s_sum[...], axis=1, keepdims=True)
        tps = jnp.sum(s_tps[...], axis=1, keepdims=True)
        ti0 = s_ti0[:, :1]
        tid = jnp.where(ssum < 1e-30, ti0, tiv)
        tid_out[...] = jnp.broadcast_to(tid, (RB, 128))
        tps_out[...] = jnp.broadcast_to(tps, (RB, 128))


def _tail_kernel(tid_ref, tps_ref, did_ref, dp_ref, rnd_ref,
                 tok_out, idx_out):
    tid = tid_ref[...]
    tps = tps_ref[...]
    did = did_ref[...]
    dp = dp_ref[...]
    rnd = rnd_ref[...]

    ratio = jnp.minimum(tps / dp, 1.0)
    acc = (rnd < ratio).astype(jnp.float32)

    r_i = jax.lax.broadcasted_iota(jnp.int32, (16, 16), 0)
    c_i = jax.lax.broadcasted_iota(jnp.int32, (16, 16), 1)
    lt = (r_i <= c_i).astype(jnp.float32)
    cs = jnp.dot(acc, lt, preferred_element_type=jnp.float32)

    lane = jax.lax.broadcasted_iota(jnp.int32, (32, 16), 1)
    positions = (lane + 1).astype(jnp.float32)
    mask = cs == positions
    index = jnp.sum(mask.astype(jnp.int32), axis=1, keepdims=True)

    tokens = jnp.where(mask, did, tid)
    keep = index >= lane
    tokens = jnp.where(keep, tokens, PAD_TOKEN_ID)

    tok_out[...] = tokens
    idx_out[...] = jnp.broadcast_to(index, (32, 16))


@jax.jit
def kernel(draft_ids, draft_probs, target_probs, target_indices):
    tp2d = target_probs.reshape(ROWS, V)
    ti2d = target_indices.reshape(ROWS, V)

    did_rows = jnp.concatenate(
        [draft_ids, jnp.full((B, 1), -1, jnp.int32)], axis=1).reshape(ROWS)
    dp_rows = jnp.concatenate(
        [draft_probs, jnp.zeros((B, 1), jnp.float32)], axis=1).reshape(ROWS)
    did_b = jnp.broadcast_to(did_rows[:, None], (ROWS, 128))
    dp_b = jnp.broadcast_to(dp_rows[:, None], (ROWS, 128))

    grid = (ROWS // RB, NV)
    tid_full, tps_full = pl.pallas_call(
        _scan_kernel,
        grid=grid,
        in_specs=[
            pl.BlockSpec((RB, VB), lambda r, v: (r, v)),
            pl.BlockSpec((RB, VB), lambda r, v: (r, v)),
            pl.BlockSpec((RB, 128), lambda r, v: (r, 0)),
            pl.BlockSpec((RB, 128), lambda r, v: (r, 0)),
        ],
        out_specs=[
            pl.BlockSpec((RB, 128), lambda r, v: (r, 0)),
            pl.BlockSpec((RB, 128), lambda r, v: (r, 0)),
        ],
        out_shape=[
            jax.ShapeDtypeStruct((ROWS, 128), jnp.int32),
            jax.ShapeDtypeStruct((ROWS, 128), jnp.float32),
        ],
        scratch_shapes=[
            pltpu.VMEM((RB, VB), jnp.float32),
            pltpu.VMEM((RB, VB), jnp.int32),
            pltpu.VMEM((RB, VB), jnp.int32),
            pltpu.VMEM((RB, VB), jnp.float32),
            pltpu.VMEM((RB, VB), jnp.float32),
            pltpu.VMEM((RB, 128), jnp.int32),
        ],
        compiler_params=pltpu.CompilerParams(
            dimension_semantics=("arbitrary", "arbitrary")),
    )(tp2d, ti2d, did_b, dp_b)

    target_ids = tid_full[:, 0].reshape(B, K + 1)
    tp_sel = tps_full[:, 0].reshape(B, K + 1)[:, :K]

    tid_p = jnp.pad(target_ids, ((0, 0), (0, 16 - (K + 1))))
    tps_p = jnp.pad(tp_sel, ((0, 0), (0, 16 - K)))
    did_p = jnp.pad(draft_ids, ((0, 0), (0, 16 - K)))
    dp_p = jnp.pad(draft_probs, ((0, 0), (0, 16 - K)), constant_values=1.0)
    rnd = jax.random.uniform(jax.random.key(42), (B, K), dtype=jnp.float32)
    rnd_p = jnp.pad(rnd, ((0, 0), (0, 16 - K)), constant_values=2.0)

    tokens16, idx16 = pl.pallas_call(
        _tail_kernel,
        out_shape=[
            jax.ShapeDtypeStruct((B, 16), jnp.int32),
            jax.ShapeDtypeStruct((B, 16), jnp.int32),
        ],
    )(tid_p, tps_p, did_p, dp_p, rnd_p)

    tokens = tokens16[:, :K + 1]
    index = idx16[:, :1]
    return (tokens, index)


# speedup vs baseline: 2.7542x; 2.3993x over previous
import jax
import jax.numpy as jnp
from jax.experimental import pallas as pl
from jax.experimental.pallas import tpu as pltpu

B, K, V = 32, 8, 100000
PAD_TOKEN_ID = 0
ROWS = B * (K + 1)
RB = 8
LT = 128
FULL = V // LT
REM = V - FULL * LT
UNROLL = 4
NLOOP = (FULL - 1) // UNROLL
IBIG = 2**31 - 1


def _scan_kernel(tp_ref, ti_ref, did_ref, dp_ref, tid_out, tps_out):
    did = did_ref[:, :1]
    dp = dp_ref[:, :1]

    def tile(t):
        tp = tp_ref[:, pl.ds(t * LT, LT)]
        ti = ti_ref[:, pl.ds(t * LT, LT)]
        eq = ti == did
        c = jnp.maximum(tp - jnp.where(eq, dp, 0.0), 0.0)
        msel = jnp.where(eq, tp, 0.0)
        return ti, c, msel

    def update(state, t, ti, c, msel):
        s_max, s_vt, s_ti, s_sum, s_tps = state
        s_sum = s_sum + c
        s_tps = s_tps + msel
        upd = c > s_max
        s_max = jnp.where(upd, c, s_max)
        s_vt = jnp.where(upd, t, s_vt)
        s_ti = jnp.where(upd, ti, s_ti)
        return (s_max, s_vt, s_ti, s_sum, s_tps)

    ti0, c0, msel0 = tile(0)
    state = (c0, jnp.zeros((RB, LT), jnp.int32), ti0, c0, msel0)

    def body(i, state):
        for j in range(UNROLL):
            t = 1 + i * UNROLL + j
            ti, c, msel = tile(t)
            state = update(state, t, ti, c, msel)
        return state

    state = jax.lax.fori_loop(0, NLOOP, body, state)

    lane = jax.lax.broadcasted_iota(jnp.int32, (RB, LT), 1)
    tp_r = tp_ref[:, pl.ds(FULL * LT, REM)]
    ti_r = ti_ref[:, pl.ds(FULL * LT, REM)]
    eq_r = ti_r == did
    c_r = jnp.maximum(tp_r - jnp.where(eq_r, dp, 0.0), 0.0)
    msel_r = jnp.where(eq_r, tp_r, 0.0)
    zf = jnp.zeros((RB, LT - REM), jnp.float32)
    zi = jnp.zeros((RB, LT - REM), jnp.int32)
    c = jnp.concatenate([c_r, zf], axis=1)
    msel = jnp.concatenate([msel_r, zf], axis=1)
    ti = jnp.concatenate([ti_r, zi], axis=1)
    s_max, s_vt, s_ti, s_sum, s_tps = update(state, FULL, ti, c, msel)

    m = jnp.max(s_max, axis=1, keepdims=True)
    colg = s_vt * LT + lane
    idx = jnp.min(jnp.where(s_max == m, colg, IBIG), axis=1, keepdims=True)
    tiv = jnp.sum(jnp.where(colg == idx, s_ti, 0), axis=1, keepdims=True)
    ssum = jnp.sum(s_sum, axis=1, keepdims=True)
    tps = jnp.sum(s_tps, axis=1, keepdims=True)
    ti0c = jnp.sum(jnp.where(colg == 0, s_ti, 0), axis=1, keepdims=True)
    tid = jnp.where(ssum < 1e-30, ti0c, tiv)
    tid_out[...] = jnp.broadcast_to(tid, (RB, 128))
    tps_out[...] = jnp.broadcast_to(tps, (RB, 128))


def _tail_kernel(tid_ref, tps_ref, did_ref, dp_ref, rnd_ref,
                 tok_out, idx_out):
    tid = tid_ref[...]
    tps = tps_ref[...]
    did = did_ref[...]
    dp = dp_ref[...]
    rnd = rnd_ref[...]

    ratio = jnp.minimum(tps / dp, 1.0)
    acc = (rnd < ratio).astype(jnp.float32)

    r_i = jax.lax.broadcasted_iota(jnp.int32, (16, 16), 0)
    c_i = jax.lax.broadcasted_iota(jnp.int32, (16, 16), 1)
    lt = (r_i <= c_i).astype(jnp.float32)
    cs = jnp.dot(acc, lt, preferred_element_type=jnp.float32)

    lane = jax.lax.broadcasted_iota(jnp.int32, (32, 16), 1)
    positions = (lane + 1).astype(jnp.float32)
    mask = cs == positions
    index = jnp.sum(mask.astype(jnp.int32), axis=1, keepdims=True)

    tokens = jnp.where(mask, did, tid)
    keep = index >= lane
    tokens = jnp.where(keep, tokens, PAD_TOKEN_ID)

    tok_out[...] = tokens
    idx_out[...] = jnp.broadcast_to(index, (32, 16))


@jax.jit
def kernel(draft_ids, draft_probs, target_probs, target_indices):
    tp2d = target_probs.reshape(ROWS, V)
    ti2d = target_indices.reshape(ROWS, V)

    did_rows = jnp.concatenate(
        [draft_ids, jnp.full((B, 1), -1, jnp.int32)], axis=1).reshape(ROWS)
    dp_rows = jnp.concatenate(
        [draft_probs, jnp.zeros((B, 1), jnp.float32)], axis=1).reshape(ROWS)
    did_b = jnp.broadcast_to(did_rows[:, None], (ROWS, 128))
    dp_b = jnp.broadcast_to(dp_rows[:, None], (ROWS, 128))

    grid = (ROWS // RB,)
    tid_full, tps_full = pl.pallas_call(
        _scan_kernel,
        grid=grid,
        in_specs=[
            pl.BlockSpec((RB, V), lambda r: (r, 0)),
            pl.BlockSpec((RB, V), lambda r: (r, 0)),
            pl.BlockSpec((RB, 128), lambda r: (r, 0)),
            pl.BlockSpec((RB, 128), lambda r: (r, 0)),
        ],
        out_specs=[
            pl.BlockSpec((RB, 128), lambda r: (r, 0)),
            pl.BlockSpec((RB, 128), lambda r: (r, 0)),
        ],
        out_shape=[
            jax.ShapeDtypeStruct((ROWS, 128), jnp.int32),
            jax.ShapeDtypeStruct((ROWS, 128), jnp.float32),
        ],
        compiler_params=pltpu.CompilerParams(
            dimension_semantics=("arbitrary",)),
    )(tp2d, ti2d, did_b, dp_b)

    target_ids = tid_full[:, 0].reshape(B, K + 1)
    tp_sel = tps_full[:, 0].reshape(B, K + 1)[:, :K]

    tid_p = jnp.pad(target_ids, ((0, 0), (0, 16 - (K + 1))))
    tps_p = jnp.pad(tp_sel, ((0, 0), (0, 16 - K)))
    did_p = jnp.pad(draft_ids, ((0, 0), (0, 16 - K)))
    dp_p = jnp.pad(draft_probs, ((0, 0), (0, 16 - K)), constant_values=1.0)
    rnd = jax.random.uniform(jax.random.key(42), (B, K), dtype=jnp.float32)
    rnd_p = jnp.pad(rnd, ((0, 0), (0, 16 - K)), constant_values=2.0)

    tokens16, idx16 = pl.pallas_call(
        _tail_kernel,
        out_shape=[
            jax.ShapeDtypeStruct((B, 16), jnp.int32),
            jax.ShapeDtypeStruct((B, 16), jnp.int32),
        ],
    )(tid_p, tps_p, did_p, dp_p, rnd_p)

    tokens = tokens16[:, :K + 1]
    index = idx16[:, :1]
    return (tokens, index)


# speedup vs baseline: 4.5223x; 1.6419x over previous
import jax
import jax.numpy as jnp
from jax.experimental import pallas as pl
from jax.experimental.pallas import tpu as pltpu

B, K, V = 32, 8, 100000
PAD_TOKEN_ID = 0
KP = K + 1
LT = 128
FULL = V // LT
REM = V - FULL * LT
UNROLL = 4
NLOOP = (FULL - 1) // UNROLL
IBIG = 2**31 - 1


def _scan_kernel(tp_ref, ti_ref, did_ref, dp_ref, tid_out, tps_out):
    did = did_ref[0, :, :1]
    dp = dp_ref[0, :, :1]

    def tile(t):
        tp = tp_ref[0, :, pl.ds(t * LT, LT)]
        ti = ti_ref[0, :, pl.ds(t * LT, LT)]
        eq = ti == did
        c = jnp.maximum(tp - jnp.where(eq, dp, 0.0), 0.0)
        msel = jnp.where(eq, tp, 0.0)
        return ti, c, msel

    def update(state, t, ti, c, msel):
        s_max, s_vt, s_ti, s_sum, s_tps = state
        s_sum = s_sum + c
        s_tps = s_tps + msel
        upd = c > s_max
        s_max = jnp.where(upd, c, s_max)
        s_vt = jnp.where(upd, t, s_vt)
        s_ti = jnp.where(upd, ti, s_ti)
        return (s_max, s_vt, s_ti, s_sum, s_tps)

    ti0, c0, msel0 = tile(0)
    state = (c0, jnp.zeros((KP, LT), jnp.int32), ti0, c0, msel0)

    def body(i, state):
        for j in range(UNROLL):
            t = 1 + i * UNROLL + j
            ti, c, msel = tile(t)
            state = update(state, t, ti, c, msel)
        return state

    state = jax.lax.fori_loop(0, NLOOP, body, state)

    lane = jax.lax.broadcasted_iota(jnp.int32, (KP, LT), 1)
    tp_r = tp_ref[0, :, pl.ds(FULL * LT, REM)]
    ti_r = ti_ref[0, :, pl.ds(FULL * LT, REM)]
    eq_r = ti_r == did
    c_r = jnp.maximum(tp_r - jnp.where(eq_r, dp, 0.0), 0.0)
    msel_r = jnp.where(eq_r, tp_r, 0.0)
    zf = jnp.zeros((KP, LT - REM), jnp.float32)
    zi = jnp.zeros((KP, LT - REM), jnp.int32)
    c = jnp.concatenate([c_r, zf], axis=1)
    msel = jnp.concatenate([msel_r, zf], axis=1)
    ti = jnp.concatenate([ti_r, zi], axis=1)
    s_max, s_vt, s_ti, s_sum, s_tps = update(state, FULL, ti, c, msel)

    m = jnp.max(s_max, axis=1, keepdims=True)
    colg = s_vt * LT + lane
    idx = jnp.min(jnp.where(s_max == m, colg, IBIG), axis=1, keepdims=True)
    tiv = jnp.sum(jnp.where(colg == idx, s_ti, 0), axis=1, keepdims=True)
    ssum = jnp.sum(s_sum, axis=1, keepdims=True)
    tps = jnp.sum(s_tps, axis=1, keepdims=True)
    ti0c = jnp.sum(jnp.where(colg == 0, s_ti, 0), axis=1, keepdims=True)
    tid = jnp.where(ssum < 1e-30, ti0c, tiv)
    tid_out[0, :, :] = jnp.broadcast_to(tid, (KP, 128))
    tps_out[0, :, :] = jnp.broadcast_to(tps, (KP, 128))


def _tail_kernel(tid_ref, tps_ref, did_ref, dp_ref, rnd_ref,
                 tok_out, idx_out):
    tid = tid_ref[...]
    tps = tps_ref[...]
    did = did_ref[...]
    dp = dp_ref[...]
    rnd = rnd_ref[...]

    ratio = jnp.minimum(tps / dp, 1.0)
    acc = (rnd < ratio).astype(jnp.float32)

    r_i = jax.lax.broadcasted_iota(jnp.int32, (16, 16), 0)
    c_i = jax.lax.broadcasted_iota(jnp.int32, (16, 16), 1)
    lt = (r_i <= c_i).astype(jnp.float32)
    cs = jnp.dot(acc, lt, preferred_element_type=jnp.float32)

    lane = jax.lax.broadcasted_iota(jnp.int32, (32, 16), 1)
    positions = (lane + 1).astype(jnp.float32)
    mask = cs == positions
    index = jnp.sum(mask.astype(jnp.int32), axis=1, keepdims=True)

    tokens = jnp.where(mask, did, tid)
    keep = index >= lane
    tokens = jnp.where(keep, tokens, PAD_TOKEN_ID)

    tok_out[...] = tokens
    idx_out[...] = jnp.broadcast_to(index, (32, 16))


@jax.jit
def kernel(draft_ids, draft_probs, target_probs, target_indices):
    did_rows = jnp.concatenate(
        [draft_ids, jnp.full((B, 1), -1, jnp.int32)], axis=1)
    dp_rows = jnp.concatenate(
        [draft_probs, jnp.zeros((B, 1), jnp.float32)], axis=1)
    did_b = jnp.broadcast_to(did_rows[:, :, None], (B, KP, 128))
    dp_b = jnp.broadcast_to(dp_rows[:, :, None], (B, KP, 128))

    grid = (B,)
    tid_full, tps_full = pl.pallas_call(
        _scan_kernel,
        grid=grid,
        in_specs=[
            pl.BlockSpec((1, KP, V), lambda b: (b, 0, 0)),
            pl.BlockSpec((1, KP, V), lambda b: (b, 0, 0)),
            pl.BlockSpec((1, KP, 128), lambda b: (b, 0, 0)),
            pl.BlockSpec((1, KP, 128), lambda b: (b, 0, 0)),
        ],
        out_specs=[
            pl.BlockSpec((1, KP, 128), lambda b: (b, 0, 0)),
            pl.BlockSpec((1, KP, 128), lambda b: (b, 0, 0)),
        ],
        out_shape=[
            jax.ShapeDtypeStruct((B, KP, 128), jnp.int32),
            jax.ShapeDtypeStruct((B, KP, 128), jnp.float32),
        ],
        compiler_params=pltpu.CompilerParams(
            dimension_semantics=("arbitrary",)),
    )(target_probs, target_indices, did_b, dp_b)

    target_ids = tid_full[:, :, 0]
    tp_sel = tps_full[:, :, 0][:, :K]

    tid_p = jnp.pad(target_ids, ((0, 0), (0, 16 - KP)))
    tps_p = jnp.pad(tp_sel, ((0, 0), (0, 16 - K)))
    did_p = jnp.pad(draft_ids, ((0, 0), (0, 16 - K)))
    dp_p = jnp.pad(draft_probs, ((0, 0), (0, 16 - K)), constant_values=1.0)
    rnd = jax.random.uniform(jax.random.key(42), (B, K), dtype=jnp.float32)
    rnd_p = jnp.pad(rnd, ((0, 0), (0, 16 - K)), constant_values=2.0)

    tokens16, idx16 = pl.pallas_call(
        _tail_kernel,
        out_shape=[
            jax.ShapeDtypeStruct((B, 16), jnp.int32),
            jax.ShapeDtypeStruct((B, 16), jnp.int32),
        ],
    )(tid_p, tps_p, did_p, dp_p, rnd_p)

    tokens = tokens16[:, :KP]
    index = idx16[:, :1]
    return (tokens, index)


# speedup vs baseline: 4.6672x; 1.0320x over previous
import jax
import jax.numpy as jnp
from jax.experimental import pallas as pl
from jax.experimental.pallas import tpu as pltpu

B, K, V = 32, 8, 100000
PAD_TOKEN_ID = 0
KP = K + 1
LT = 128
FULL = V // LT
REM = V - FULL * LT
UNROLL = 4
NLOOP = (FULL - 1) // UNROLL
IBIG = 2**31 - 1


def _scan_kernel(tp_ref, ti_ref, did_ref, dp_ref, tid_out, tps_out):
    did = did_ref[0, :, :1]
    dp = dp_ref[0, :, :1]

    def tile(t):
        tp = tp_ref[0, :, pl.ds(t * LT, LT)]
        ti = ti_ref[0, :, pl.ds(t * LT, LT)]
        eq = ti == did
        c = jnp.maximum(tp - jnp.where(eq, dp, 0.0), 0.0)
        msel = jnp.where(eq, tp, 0.0)
        return ti, c, msel

    def update(state, t, ti, c, msel):
        s_max, s_vt, s_ti, s_sum, s_tps = state
        s_sum = s_sum + c
        s_tps = s_tps + msel
        upd = c > s_max
        s_max = jnp.where(upd, c, s_max)
        s_vt = jnp.where(upd, t, s_vt)
        s_ti = jnp.where(upd, ti, s_ti)
        return (s_max, s_vt, s_ti, s_sum, s_tps)

    ti0, c0, msel0 = tile(0)
    state = (c0, jnp.zeros((K, LT), jnp.int32), ti0, c0, msel0)

    def body(i, state):
        for j in range(UNROLL):
            t = 1 + i * UNROLL + j
            state = update(state, t, *tile(t))
        return state

    state = jax.lax.fori_loop(0, NLOOP, body, state)

    lane = jax.lax.broadcasted_iota(jnp.int32, (K, LT), 1)
    tp_r = tp_ref[0, :, pl.ds(FULL * LT, REM)]
    ti_r = ti_ref[0, :, pl.ds(FULL * LT, REM)]
    eq_r = ti_r == did
    c_r = jnp.maximum(tp_r - jnp.where(eq_r, dp, 0.0), 0.0)
    msel_r = jnp.where(eq_r, tp_r, 0.0)
    zf = jnp.zeros((K, LT - REM), jnp.float32)
    zi = jnp.zeros((K, LT - REM), jnp.int32)
    c = jnp.concatenate([c_r, zf], axis=1)
    msel = jnp.concatenate([msel_r, zf], axis=1)
    ti = jnp.concatenate([ti_r, zi], axis=1)
    s_max, s_vt, s_ti, s_sum, s_tps = update(state, FULL, ti, c, msel)

    m = jnp.max(s_max, axis=1, keepdims=True)
    colg = s_vt * LT + lane
    idx = jnp.min(jnp.where(s_max == m, colg, IBIG), axis=1, keepdims=True)
    tiv = jnp.sum(jnp.where(colg == idx, s_ti, 0), axis=1, keepdims=True)
    ssum = jnp.sum(s_sum, axis=1, keepdims=True)
    tps = jnp.sum(s_tps, axis=1, keepdims=True)
    ti0c = jnp.sum(jnp.where(colg == 0, s_ti, 0), axis=1, keepdims=True)
    tid = jnp.where(ssum < 1e-30, ti0c, tiv)
    tid_out[0, :, :] = jnp.broadcast_to(tid, (K, 128))
    tps_out[0, :, :] = jnp.broadcast_to(tps, (K, 128))


def _argmax_kernel(tp_ref, ti_ref, tid_out):
    def tile(t):
        return (tp_ref[:, pl.ds(t * LT, LT)],
                ti_ref[:, pl.ds(t * LT, LT)])

    def update(state, t, tp, ti):
        r_max, r_vt, r_ti = state
        upd = tp > r_max
        r_max = jnp.where(upd, tp, r_max)
        r_vt = jnp.where(upd, t, r_vt)
        r_ti = jnp.where(upd, ti, r_ti)
        return (r_max, r_vt, r_ti)

    tp0, ti0 = tile(0)
    state = (tp0, jnp.zeros((8, LT), jnp.int32), ti0)

    def body(i, state):
        for j in range(UNROLL):
            t = 1 + i * UNROLL + j
            state = update(state, t, *tile(t))
        return state

    state = jax.lax.fori_loop(0, NLOOP, body, state)

    tp_r = tp_ref[:, pl.ds(FULL * LT, REM)]
    ti_r = ti_ref[:, pl.ds(FULL * LT, REM)]
    zf = jnp.zeros((8, LT - REM), jnp.float32)
    zi = jnp.zeros((8, LT - REM), jnp.int32)
    tp = jnp.concatenate([tp_r, zf], axis=1)
    ti = jnp.concatenate([ti_r, zi], axis=1)
    r_max, r_vt, r_ti = update(state, FULL, tp, ti)

    lane = jax.lax.broadcasted_iota(jnp.int32, (8, LT), 1)
    m = jnp.max(r_max, axis=1, keepdims=True)
    colg = r_vt * LT + lane
    idx = jnp.min(jnp.where(r_max == m, colg, IBIG), axis=1, keepdims=True)
    tiv = jnp.sum(jnp.where(colg == idx, r_ti, 0), axis=1, keepdims=True)
    tid_out[...] = jnp.broadcast_to(tiv, (8, 128))


def _tail_kernel(tid_ref, tps_ref, did_ref, dp_ref, rnd_ref,
                 tok_out, idx_out):
    tid = tid_ref[...]
    tps = tps_ref[...]
    did = did_ref[...]
    dp = dp_ref[...]
    rnd = rnd_ref[...]

    ratio = jnp.minimum(tps / dp, 1.0)
    acc = (rnd < ratio).astype(jnp.float32)

    r_i = jax.lax.broadcasted_iota(jnp.int32, (16, 16), 0)
    c_i = jax.lax.broadcasted_iota(jnp.int32, (16, 16), 1)
    lt = (r_i <= c_i).astype(jnp.float32)
    cs = jnp.dot(acc, lt, preferred_element_type=jnp.float32)

    lane = jax.lax.broadcasted_iota(jnp.int32, (32, 16), 1)
    positions = (lane + 1).astype(jnp.float32)
    mask = cs == positions
    index = jnp.sum(mask.astype(jnp.int32), axis=1, keepdims=True)

    tokens = jnp.where(mask, did, tid)
    keep = index >= lane
    tokens = jnp.where(keep, tokens, PAD_TOKEN_ID)

    tok_out[...] = tokens
    idx_out[...] = jnp.broadcast_to(index, (32, 16))


@jax.jit
def kernel(draft_ids, draft_probs, target_probs, target_indices):
    did_b = jnp.broadcast_to(draft_ids[:, :, None], (B, K, 128))
    dp_b = jnp.broadcast_to(draft_probs[:, :, None], (B, K, 128))

    tid_full, tps_full = pl.pallas_call(
        _scan_kernel,
        grid=(B,),
        in_specs=[
            pl.BlockSpec((1, K, V), lambda b: (b, 0, 0)),
            pl.BlockSpec((1, K, V), lambda b: (b, 0, 0)),
            pl.BlockSpec((1, K, 128), lambda b: (b, 0, 0)),
            pl.BlockSpec((1, K, 128), lambda b: (b, 0, 0)),
        ],
        out_specs=[
            pl.BlockSpec((1, K, 128), lambda b: (b, 0, 0)),
            pl.BlockSpec((1, K, 128), lambda b: (b, 0, 0)),
        ],
        out_shape=[
            jax.ShapeDtypeStruct((B, K, 128), jnp.int32),
            jax.ShapeDtypeStruct((B, K, 128), jnp.float32),
        ],
        compiler_params=pltpu.CompilerParams(
            dimension_semantics=("arbitrary",)),
    )(target_probs, target_indices, did_b, dp_b)

    tp8 = target_probs[:, K, :]
    ti8 = target_indices[:, K, :]
    tid8 = pl.pallas_call(
        _argmax_kernel,
        grid=(B // 8,),
        in_specs=[
            pl.BlockSpec((8, V), lambda r: (r, 0)),
            pl.BlockSpec((8, V), lambda r: (r, 0)),
        ],
        out_specs=pl.BlockSpec((8, 128), lambda r: (r, 0)),
        out_shape=jax.ShapeDtypeStruct((B, 128), jnp.int32),
        compiler_params=pltpu.CompilerParams(
            dimension_semantics=("arbitrary",)),
    )(tp8, ti8)

    target_ids = jnp.concatenate(
        [tid_full[:, :, 0], tid8[:, :1]], axis=1)
    tp_sel = tps_full[:, :, 0]

    tid_p = jnp.pad(target_ids, ((0, 0), (0, 16 - KP)))
    tps_p = jnp.pad(tp_sel, ((0, 0), (0, 16 - K)))
    did_p = jnp.pad(draft_ids, ((0, 0), (0, 16 - K)))
    dp_p = jnp.pad(draft_probs, ((0, 0), (0, 16 - K)), constant_values=1.0)
    rnd = jax.random.uniform(jax.random.key(42), (B, K), dtype=jnp.float32)
    rnd_p = jnp.pad(rnd, ((0, 0), (0, 16 - K)), constant_values=2.0)

    tokens16, idx16 = pl.pallas_call(
        _tail_kernel,
        out_shape=[
            jax.ShapeDtypeStruct((B, 16), jnp.int32),
            jax.ShapeDtypeStruct((B, 16), jnp.int32),
        ],
    )(tid_p, tps_p, did_p, dp_p, rnd_p)

    tokens = tokens16[:, :KP]
    index = idx16[:, :1]
    return (tokens, index)
